# baseline (speedup 1.0000x reference)
import functools

import jax
import jax.numpy as jnp
from jax import lax
from jax.experimental import pallas as pl
from jax.experimental.pallas import tpu as pltpu
from jax.experimental.pallas import tpu_sc as plsc

F_FIELDS = 26
VOCAB = 100000
BATCH = 16384
D_CAT = 32
D_NUM = 64
D_QUAL = 24
D_TOT = D_CAT + D_NUM + D_QUAL

NC, NS = 2, 16
NW = NC * NS
B_W = BATCH // NW
CHUNK = 128
NCH = B_W // CHUNK
LANES = 16


def _sc_gather_body(idx_hbm, tab_hbm, out_hbm, idx_v, acc_v, buf_v, sem, isem):
  wid = lax.axis_index("s") * NC + lax.axis_index("c")
  base = wid * B_W

  for f in range(F_FIELDS):
    pltpu.async_copy(idx_hbm.at[f, wid], idx_v.at[f], isem)

  zeros = jnp.zeros((LANES,), jnp.float32)
  @pl.loop(0, B_W, unroll=8)
  def _zero(r):
    for h in range(D_CAT // LANES):
      acc_v[r, pl.ds(h * LANES, LANES)] = zeros

  for f in range(F_FIELDS):
    pltpu.make_async_copy(idx_hbm.at[f, wid], idx_v.at[f], isem).wait()

  def fire(f, slot):
    for c in range(NCH):
      pltpu.async_copy(
          tab_hbm.at[f].at[idx_v.at[f, pl.ds(c * CHUNK, CHUNK)]],
          buf_v.at[slot].at[pl.ds(c * CHUNK, CHUNK)],
          sem.at[slot],
      )

  def drain(f, slot):
    for c in range(NCH):
      pltpu.make_async_copy(
          tab_hbm.at[f].at[idx_v.at[f, pl.ds(c * CHUNK, CHUNK)]],
          buf_v.at[slot].at[pl.ds(c * CHUNK, CHUNK)],
          sem.at[slot],
      ).wait()

  fire(0, 0)
  for f in range(F_FIELDS):
    slot = f % 2
    if f + 1 < F_FIELDS:
      fire(f + 1, 1 - slot)
    drain(f, slot)
    @pl.loop(0, B_W, unroll=8)
    def _acc(r):
      for h in range(D_CAT // LANES):
        v = buf_v[slot, r, pl.ds(h * LANES, LANES)]
        plsc.addupdate(acc_v.at[r, pl.ds(h * LANES, LANES)], v)

  pltpu.sync_copy(acc_v, out_hbm.at[pl.ds(base, B_W)])


def _sc_gather(idx3, tables):
  mesh = plsc.VectorSubcoreMesh(
      core_axis_name="c", subcore_axis_name="s", num_cores=NC, num_subcores=NS)
  return pl.kernel(
      _sc_gather_body,
      out_type=jax.ShapeDtypeStruct((BATCH, D_CAT), jnp.float32),
      mesh=mesh,
      scratch_types=[
          pltpu.VMEM((F_FIELDS, B_W), jnp.int32),
          pltpu.VMEM((B_W, D_CAT), jnp.float32),
          pltpu.VMEM((2, B_W, D_CAT), jnp.float32),
          pltpu.SemaphoreType.DMA((2,)),
          pltpu.SemaphoreType.DMA,
      ],
      compiler_params=pltpu.CompilerParams(use_tc_tiling_on_sc=False),
  )(idx3, tables)


def _tc_body(ecats_ref, nums_ref, quals_ref, W1_ref, b1_ref, W2_ref,
             b2_ref, Wq_ref, bq_ref, gamma_ref, beta_ref, out_ref):
  ec = ecats_ref[...]
  h = jnp.maximum(
      jnp.dot(nums_ref[...], W1_ref[...],
              preferred_element_type=jnp.float32) + b1_ref[...], 0.0)
  e_num = jnp.dot(h, W2_ref[...],
                  preferred_element_type=jnp.float32) + b2_ref[...]
  e_qual = jnp.dot(quals_ref[...], Wq_ref[...],
                   preferred_element_type=jnp.float32) + bq_ref[...]

  s = (jnp.sum(ec, -1, keepdims=True) + jnp.sum(e_num, -1, keepdims=True)
       + jnp.sum(e_qual, -1, keepdims=True))
  sq = (jnp.sum(ec * ec, -1, keepdims=True)
        + jnp.sum(e_num * e_num, -1, keepdims=True)
        + jnp.sum(e_qual * e_qual, -1, keepdims=True))
  mu = s * (1.0 / D_TOT)
  var = sq * (1.0 / D_TOT) - mu * mu
  inv = lax.rsqrt(var + 1e-5)

  g = gamma_ref[...]
  bt = beta_ref[...]
  out_ref[:, 0:D_CAT] = ((ec - mu) * inv) * g[:, 0:D_CAT] + bt[:, 0:D_CAT]
  out_ref[:, D_CAT:D_CAT + D_NUM] = (
      ((e_num - mu) * inv) * g[:, D_CAT:D_CAT + D_NUM]
      + bt[:, D_CAT:D_CAT + D_NUM])
  out_ref[:, D_CAT + D_NUM:D_TOT] = (
      ((e_qual - mu) * inv) * g[:, D_CAT + D_NUM:D_TOT]
      + bt[:, D_CAT + D_NUM:D_TOT])


def _tc_dense(e_cats, nums, quals, W1, b1, W2, b2, Wq, bq, gamma, beta):
  BLK = 2048
  grid = (BATCH // BLK,)
  full = lambda shape: pl.BlockSpec(shape, lambda i: (0, 0))
  return pl.pallas_call(
      _tc_body,
      grid=grid,
      in_specs=[
          pl.BlockSpec((BLK, D_CAT), lambda i: (i, 0)),
          pl.BlockSpec((BLK, 64), lambda i: (i, 0)),
          pl.BlockSpec((BLK, D_QUAL), lambda i: (i, 0)),
          full((64, 64)),
          full((1, 64)),
          full((64, D_NUM)),
          full((1, D_NUM)),
          full((D_QUAL, D_QUAL)),
          full((1, D_QUAL)),
          full((1, D_TOT)),
          full((1, D_TOT)),
      ],
      out_specs=pl.BlockSpec((BLK, D_TOT), lambda i: (i, 0)),
      out_shape=jax.ShapeDtypeStruct((BATCH, D_TOT), jnp.float32),
  )(e_cats, nums, quals, W1, b1, W2, b2, Wq, bq, gamma, beta)


def kernel(cats, nums, quals, tables, W1, b1, W2, b2, Wq, bq, gamma, beta):
  idx3 = cats.astype(jnp.int32).reshape(F_FIELDS, NW, B_W)
  e_cats = _sc_gather(idx3, tables)
  return _tc_dense(
      e_cats, nums, quals, W1, b1.reshape(1, -1), W2, b2.reshape(1, -1),
      Wq, bq.reshape(1, -1), gamma.reshape(1, -1), beta.reshape(1, -1))


# speedup vs baseline: 3.0803x; 3.0803x over previous
import jax
import jax.numpy as jnp
from jax import lax
from jax.experimental import pallas as pl
from jax.experimental.pallas import tpu as pltpu
from jax.experimental.pallas import tpu_sc as plsc

F_FIELDS = 26
VOCAB = 100000
BATCH = 16384
D_CAT = 32
D_NUM = 64
D_QUAL = 24
D_TOT = D_CAT + D_NUM + D_QUAL

NC, NS = 2, 16
NW = NC * NS
CH = 4096
NCHK = BATCH // CH
LANES = 16


def _sc_body(cats_hbm, tab_hbm, out_hbm, row_v, acc_v, idx_v, rsem, isem):
  wid = lax.axis_index("s") * NC + lax.axis_index("c")

  pltpu.async_copy(tab_hbm.at[0, wid], row_v, rsem)
  pltpu.async_copy(cats_hbm.at[0, pl.ds(0, CH)], idx_v.at[0], isem.at[0])

  zeros = jnp.zeros((LANES,), jnp.float32)
  @pl.loop(0, BATCH // LANES, unroll=8)
  def _zero(i):
    acc_v[pl.ds(i * LANES, LANES)] = zeros

  @pl.loop(0, F_FIELDS)
  def _field(f):
    pltpu.make_async_copy(tab_hbm.at[f, wid], row_v, rsem).wait()
    for c in range(NCHK):
      s = c % 2
      if c + 1 < NCHK:
        pltpu.async_copy(cats_hbm.at[f, pl.ds((c + 1) * CH, CH)],
                         idx_v.at[1 - s], isem.at[1 - s])
      else:
        @pl.when(f + 1 < F_FIELDS)
        def _():
          pltpu.async_copy(cats_hbm.at[f + 1, pl.ds(0, CH)],
                           idx_v.at[1 - s], isem.at[1 - s])
      pltpu.make_async_copy(cats_hbm.at[f, pl.ds(c * CH, CH)],
                            idx_v.at[s], isem.at[s]).wait()
      @pl.loop(0, CH // LANES, unroll=8)
      def _gather(i):
        vi = idx_v[s, pl.ds(i * LANES, LANES)]
        g = plsc.load_gather(row_v, [vi])
        plsc.addupdate(acc_v.at[pl.ds(c * CH + i * LANES, LANES)], g)
    @pl.when(f + 1 < F_FIELDS)
    def _():
      pltpu.async_copy(tab_hbm.at[f + 1, wid], row_v, rsem)

  pltpu.sync_copy(acc_v, out_hbm.at[wid])


def _sc_gather_t(cats, tab_t):
  mesh = plsc.VectorSubcoreMesh(
      core_axis_name="c", subcore_axis_name="s", num_cores=NC, num_subcores=NS)
  return pl.kernel(
      _sc_body,
      out_type=jax.ShapeDtypeStruct((D_CAT, BATCH), jnp.float32),
      mesh=mesh,
      scratch_types=[
          pltpu.VMEM((VOCAB,), jnp.float32),
          pltpu.VMEM((BATCH,), jnp.float32),
          pltpu.VMEM((2, CH), jnp.int32),
          pltpu.SemaphoreType.DMA,
          pltpu.SemaphoreType.DMA((2,)),
      ],
      compiler_params=pltpu.CompilerParams(
          use_tc_tiling_on_sc=True, needs_layout_passes=False),
  )(cats, tab_t)


def _tc_body(ecat_ref, numsT_ref, qualsT_ref, W1_ref, b1_ref, W2_ref,
             b2_ref, Wq_ref, bq_ref, gamma_ref, beta_ref, out_ref):
  ecT = ecat_ref[...]
  cT = (((0,), (0,)), ((), ()))
  hT = jnp.maximum(
      lax.dot_general(W1_ref[...], numsT_ref[...], cT,
                      preferred_element_type=jnp.float32) + b1_ref[...], 0.0)
  enT = lax.dot_general(W2_ref[...], hT, cT,
                        preferred_element_type=jnp.float32) + b2_ref[...]
  eqT = lax.dot_general(Wq_ref[...], qualsT_ref[...], cT,
                        preferred_element_type=jnp.float32) + bq_ref[...]

  s = (jnp.sum(ecT, 0, keepdims=True) + jnp.sum(enT, 0, keepdims=True)
       + jnp.sum(eqT, 0, keepdims=True))
  sq = (jnp.sum(ecT * ecT, 0, keepdims=True)
        + jnp.sum(enT * enT, 0, keepdims=True)
        + jnp.sum(eqT * eqT, 0, keepdims=True))
  mu = s * (1.0 / D_TOT)
  var = sq * (1.0 / D_TOT) - mu * mu
  inv = lax.rsqrt(var + 1e-5)

  g = gamma_ref[...]
  bt = beta_ref[...]
  out_ref[0:D_CAT, :] = ((ecT - mu) * inv) * g[0:D_CAT] + bt[0:D_CAT]
  out_ref[D_CAT:D_CAT + D_NUM, :] = (
      ((enT - mu) * inv) * g[D_CAT:D_CAT + D_NUM] + bt[D_CAT:D_CAT + D_NUM])
  out_ref[D_CAT + D_NUM:D_TOT, :] = (
      ((eqT - mu) * inv) * g[D_CAT + D_NUM:D_TOT] + bt[D_CAT + D_NUM:D_TOT])


def _tc_dense_t(ecatT, numsT, qualsT, W1, b1, W2, b2, Wq, bq, gamma, beta):
  BLK = 2048
  grid = (BATCH // BLK,)
  full = lambda shape: pl.BlockSpec(shape, lambda i: (0, 0))
  return pl.pallas_call(
      _tc_body,
      grid=grid,
      in_specs=[
          pl.BlockSpec((D_CAT, BLK), lambda i: (0, i)),
          pl.BlockSpec((64, BLK), lambda i: (0, i)),
          pl.BlockSpec((D_QUAL, BLK), lambda i: (0, i)),
          full((64, 64)),
          full((64, 1)),
          full((64, D_NUM)),
          full((D_NUM, 1)),
          full((D_QUAL, D_QUAL)),
          full((D_QUAL, 1)),
          full((D_TOT, 1)),
          full((D_TOT, 1)),
      ],
      out_specs=pl.BlockSpec((D_TOT, BLK), lambda i: (0, i)),
      out_shape=jax.ShapeDtypeStruct((D_TOT, BATCH), jnp.float32),
  )(ecatT, numsT, qualsT, W1, b1, W2, b2, Wq, bq, gamma, beta)


def kernel(cats, nums, quals, tables, W1, b1, W2, b2, Wq, bq, gamma, beta):
  cats = cats.astype(jnp.int32)
  tab_t = tables.transpose(0, 2, 1)
  ecatT = _sc_gather_t(cats, tab_t)
  outT = _tc_dense_t(
      ecatT, nums.T, quals.T, W1, b1.reshape(-1, 1), W2, b2.reshape(-1, 1),
      Wq, bq.reshape(-1, 1), gamma.reshape(-1, 1), beta.reshape(-1, 1))
  return outT.T


# speedup vs baseline: 5.4177x; 1.7588x over previous
import jax
import jax.numpy as jnp
from jax import lax
from jax.experimental import pallas as pl
from jax.experimental.pallas import tpu as pltpu
from jax.experimental.pallas import tpu_sc as plsc

F_FIELDS = 26
VOCAB = 100000
BATCH = 16384
D_CAT = 32
D_NUM = 64
D_QUAL = 24
D_TOT = D_CAT + D_NUM + D_QUAL

NC, NS = 2, 16
NW = NC * NS
CH = 4096
NCHK = BATCH // CH
LANES = 16


def _sc_body(cats_hbm, tab_hbm, out_hbm, row_v, acc_v, idx_v, rsem, isem):
  wid = lax.axis_index("s") * NC + lax.axis_index("c")

  pltpu.async_copy(tab_hbm.at[0, wid], row_v, rsem)
  pltpu.async_copy(cats_hbm.at[0, pl.ds(0, CH)], idx_v.at[0], isem.at[0])

  zeros = jnp.zeros((LANES,), jnp.float32)
  @plsc.parallel_loop(0, BATCH // LANES, unroll=8)
  def _zero(i):
    acc_v[pl.ds(i * LANES, LANES)] = zeros

  @pl.loop(0, F_FIELDS)
  def _field(f):
    pltpu.make_async_copy(tab_hbm.at[f, wid], row_v, rsem).wait()
    for c in range(NCHK):
      s = c % 2
      if c + 1 < NCHK:
        pltpu.async_copy(cats_hbm.at[f, pl.ds((c + 1) * CH, CH)],
                         idx_v.at[1 - s], isem.at[1 - s])
      else:
        @pl.when(f + 1 < F_FIELDS)
        def _():
          pltpu.async_copy(cats_hbm.at[f + 1, pl.ds(0, CH)],
                           idx_v.at[1 - s], isem.at[1 - s])
      pltpu.make_async_copy(cats_hbm.at[f, pl.ds(c * CH, CH)],
                            idx_v.at[s], isem.at[s]).wait()
      @plsc.parallel_loop(0, CH // LANES, unroll=8)
      def _gather(i):
        vi = idx_v[s, pl.ds(i * LANES, LANES)]
        g = plsc.load_gather(row_v, [vi])
        plsc.addupdate(acc_v.at[pl.ds(c * CH + i * LANES, LANES)], g)
    @pl.when(f + 1 < F_FIELDS)
    def _():
      pltpu.async_copy(tab_hbm.at[f + 1, wid], row_v, rsem)

  pltpu.sync_copy(acc_v, out_hbm.at[wid])


def _sc_gather_t(cats, tab_t):
  mesh = plsc.VectorSubcoreMesh(
      core_axis_name="c", subcore_axis_name="s", num_cores=NC, num_subcores=NS)
  return pl.kernel(
      _sc_body,
      out_type=jax.ShapeDtypeStruct((D_CAT, BATCH), jnp.float32),
      mesh=mesh,
      scratch_types=[
          pltpu.VMEM((VOCAB,), jnp.float32),
          pltpu.VMEM((BATCH,), jnp.float32),
          pltpu.VMEM((2, CH), jnp.int32),
          pltpu.SemaphoreType.DMA,
          pltpu.SemaphoreType.DMA((2,)),
      ],
      compiler_params=pltpu.CompilerParams(
          use_tc_tiling_on_sc=True, needs_layout_passes=False),
  )(cats, tab_t)


def _tc_body(ecat_ref, numsT_ref, qualsT_ref, W1_ref, b1_ref, W2_ref,
             b2_ref, Wq_ref, bq_ref, gamma_ref, beta_ref, out_ref):
  ecT = ecat_ref[...]
  cT = (((0,), (0,)), ((), ()))
  hT = jnp.maximum(
      lax.dot_general(W1_ref[...], numsT_ref[...], cT,
                      preferred_element_type=jnp.float32) + b1_ref[...], 0.0)
  enT = lax.dot_general(W2_ref[...], hT, cT,
                        preferred_element_type=jnp.float32) + b2_ref[...]
  eqT = lax.dot_general(Wq_ref[...], qualsT_ref[...], cT,
                        preferred_element_type=jnp.float32) + bq_ref[...]

  s = (jnp.sum(ecT, 0, keepdims=True) + jnp.sum(enT, 0, keepdims=True)
       + jnp.sum(eqT, 0, keepdims=True))
  sq = (jnp.sum(ecT * ecT, 0, keepdims=True)
        + jnp.sum(enT * enT, 0, keepdims=True)
        + jnp.sum(eqT * eqT, 0, keepdims=True))
  mu = s * (1.0 / D_TOT)
  var = sq * (1.0 / D_TOT) - mu * mu
  inv = lax.rsqrt(var + 1e-5)

  g = gamma_ref[...]
  bt = beta_ref[...]
  out_ref[0:D_CAT, :] = ((ecT - mu) * inv) * g[0:D_CAT] + bt[0:D_CAT]
  out_ref[D_CAT:D_CAT + D_NUM, :] = (
      ((enT - mu) * inv) * g[D_CAT:D_CAT + D_NUM] + bt[D_CAT:D_CAT + D_NUM])
  out_ref[D_CAT + D_NUM:D_TOT, :] = (
      ((eqT - mu) * inv) * g[D_CAT + D_NUM:D_TOT] + bt[D_CAT + D_NUM:D_TOT])


def _tc_dense_t(ecatT, numsT, qualsT, W1, b1, W2, b2, Wq, bq, gamma, beta):
  BLK = 2048
  grid = (BATCH // BLK,)
  full = lambda shape: pl.BlockSpec(shape, lambda i: (0, 0))
  return pl.pallas_call(
      _tc_body,
      grid=grid,
      in_specs=[
          pl.BlockSpec((D_CAT, BLK), lambda i: (0, i)),
          pl.BlockSpec((64, BLK), lambda i: (0, i)),
          pl.BlockSpec((D_QUAL, BLK), lambda i: (0, i)),
          full((64, 64)),
          full((64, 1)),
          full((64, D_NUM)),
          full((D_NUM, 1)),
          full((D_QUAL, D_QUAL)),
          full((D_QUAL, 1)),
          full((D_TOT, 1)),
          full((D_TOT, 1)),
      ],
      out_specs=pl.BlockSpec((D_TOT, BLK), lambda i: (0, i)),
      out_shape=jax.ShapeDtypeStruct((D_TOT, BATCH), jnp.float32),
  )(ecatT, numsT, qualsT, W1, b1, W2, b2, Wq, bq, gamma, beta)


def kernel(cats, nums, quals, tables, W1, b1, W2, b2, Wq, bq, gamma, beta):
  cats = cats.astype(jnp.int32)
  tab_t = tables.transpose(0, 2, 1)
  ecatT = _sc_gather_t(cats, tab_t)
  outT = _tc_dense_t(
      ecatT, nums.T, quals.T, W1, b1.reshape(-1, 1), W2, b2.reshape(-1, 1),
      Wq, bq.reshape(-1, 1), gamma.reshape(-1, 1), beta.reshape(-1, 1))
  return outT.T


# speedup vs baseline: 5.8430x; 1.0785x over previous
import jax
import jax.numpy as jnp
from jax import lax
from jax.experimental import pallas as pl
from jax.experimental.pallas import tpu as pltpu
from jax.experimental.pallas import tpu_sc as plsc

F_FIELDS = 26
VOCAB = 100000
BATCH = 16384
D_CAT = 32
D_NUM = 64
D_QUAL = 24
D_TOT = D_CAT + D_NUM + D_QUAL

NC, NS = 2, 16
NW = NC * NS
CH = 4096
NCHK = BATCH // CH
LANES = 16


def _sc_body(cats_hbm, tab_hbm, out_hbm, row_v, acc_v, idx_v, shidx, rsem,
             isem):
  sid = lax.axis_index("s")
  wid = sid * NC + lax.axis_index("c")

  pltpu.async_copy(tab_hbm.at[0, wid], row_v, rsem)

  @pl.when(sid == 0)
  def _stage0():
    pltpu.sync_copy(cats_hbm.at[0], shidx.at[0])

  zeros = jnp.zeros((LANES,), jnp.float32)
  @plsc.parallel_loop(0, BATCH // LANES, unroll=8)
  def _zero(i):
    acc_v[pl.ds(i * LANES, LANES)] = zeros

  plsc.subcore_barrier()
  pltpu.async_copy(shidx.at[0, pl.ds(0, CH)], idx_v.at[0], isem.at[0])

  @pl.loop(0, F_FIELDS)
  def _field(f):
    fslot = lax.rem(f, 2)
    @pl.when((sid == 0) & (f + 1 < F_FIELDS))
    def _stage_next():
      pltpu.sync_copy(cats_hbm.at[f + 1], shidx.at[1 - fslot])
    pltpu.make_async_copy(tab_hbm.at[f, wid], row_v, rsem).wait()
    for c in range(NCHK):
      s = c % 2
      if c + 1 < NCHK:
        pltpu.async_copy(shidx.at[fslot, pl.ds((c + 1) * CH, CH)],
                         idx_v.at[1 - s], isem.at[1 - s])
      pltpu.make_async_copy(shidx.at[fslot, pl.ds(c * CH, CH)],
                            idx_v.at[s], isem.at[s]).wait()
      @plsc.parallel_loop(0, CH // LANES, unroll=8)
      def _gather(i):
        vi = idx_v[s, pl.ds(i * LANES, LANES)]
        g = plsc.load_gather(row_v, [vi])
        plsc.addupdate(acc_v.at[pl.ds(c * CH + i * LANES, LANES)], g)
    @pl.when(f + 1 < F_FIELDS)
    def _next_row():
      pltpu.async_copy(tab_hbm.at[f + 1, wid], row_v, rsem)
    plsc.subcore_barrier()
    @pl.when(f + 1 < F_FIELDS)
    def _next_idx():
      pltpu.async_copy(shidx.at[1 - fslot, pl.ds(0, CH)],
                       idx_v.at[NCHK % 2], isem.at[NCHK % 2])

  pltpu.sync_copy(acc_v, out_hbm.at[wid])


def _sc_gather_t(cats, tab_t):
  mesh = plsc.VectorSubcoreMesh(
      core_axis_name="c", subcore_axis_name="s", num_cores=NC, num_subcores=NS)
  return pl.kernel(
      _sc_body,
      out_type=jax.ShapeDtypeStruct((D_CAT, BATCH), jnp.float32),
      mesh=mesh,
      scratch_types=[
          pltpu.VMEM((VOCAB,), jnp.float32),
          pltpu.VMEM((BATCH,), jnp.float32),
          pltpu.VMEM((2, CH), jnp.int32),
          pltpu.VMEM_SHARED((2, BATCH), jnp.int32),
          pltpu.SemaphoreType.DMA,
          pltpu.SemaphoreType.DMA((2,)),
      ],
      compiler_params=pltpu.CompilerParams(
          use_tc_tiling_on_sc=True, needs_layout_passes=False),
  )(cats, tab_t)


def _tc_body(ecat_ref, numsT_ref, qualsT_ref, W1_ref, b1_ref, W2_ref,
             b2_ref, Wq_ref, bq_ref, gamma_ref, beta_ref, out_ref):
  ecT = ecat_ref[...]
  cT = (((0,), (0,)), ((), ()))
  hT = jnp.maximum(
      lax.dot_general(W1_ref[...], numsT_ref[...], cT,
                      preferred_element_type=jnp.float32) + b1_ref[...], 0.0)
  enT = lax.dot_general(W2_ref[...], hT, cT,
                        preferred_element_type=jnp.float32) + b2_ref[...]
  eqT = lax.dot_general(Wq_ref[...], qualsT_ref[...], cT,
                        preferred_element_type=jnp.float32) + bq_ref[...]

  s = (jnp.sum(ecT, 0, keepdims=True) + jnp.sum(enT, 0, keepdims=True)
       + jnp.sum(eqT, 0, keepdims=True))
  sq = (jnp.sum(ecT * ecT, 0, keepdims=True)
        + jnp.sum(enT * enT, 0, keepdims=True)
        + jnp.sum(eqT * eqT, 0, keepdims=True))
  mu = s * (1.0 / D_TOT)
  var = sq * (1.0 / D_TOT) - mu * mu
  inv = lax.rsqrt(var + 1e-5)

  g = gamma_ref[...]
  bt = beta_ref[...]
  out_ref[0:D_CAT, :] = ((ecT - mu) * inv) * g[0:D_CAT] + bt[0:D_CAT]
  out_ref[D_CAT:D_CAT + D_NUM, :] = (
      ((enT - mu) * inv) * g[D_CAT:D_CAT + D_NUM] + bt[D_CAT:D_CAT + D_NUM])
  out_ref[D_CAT + D_NUM:D_TOT, :] = (
      ((eqT - mu) * inv) * g[D_CAT + D_NUM:D_TOT] + bt[D_CAT + D_NUM:D_TOT])


def _tc_dense_t(ecatT, numsT, qualsT, W1, b1, W2, b2, Wq, bq, gamma, beta):
  BLK = 2048
  grid = (BATCH // BLK,)
  full = lambda shape: pl.BlockSpec(shape, lambda i: (0, 0))
  return pl.pallas_call(
      _tc_body,
      grid=grid,
      in_specs=[
          pl.BlockSpec((D_CAT, BLK), lambda i: (0, i)),
          pl.BlockSpec((64, BLK), lambda i: (0, i)),
          pl.BlockSpec((D_QUAL, BLK), lambda i: (0, i)),
          full((64, 64)),
          full((64, 1)),
          full((64, D_NUM)),
          full((D_NUM, 1)),
          full((D_QUAL, D_QUAL)),
          full((D_QUAL, 1)),
          full((D_TOT, 1)),
          full((D_TOT, 1)),
      ],
      out_specs=pl.BlockSpec((D_TOT, BLK), lambda i: (0, i)),
      out_shape=jax.ShapeDtypeStruct((D_TOT, BATCH), jnp.float32),
  )(ecatT, numsT, qualsT, W1, b1, W2, b2, Wq, bq, gamma, beta)


def kernel(cats, nums, quals, tables, W1, b1, W2, b2, Wq, bq, gamma, beta):
  cats = cats.astype(jnp.int32)
  tab_t = tables.transpose(0, 2, 1)
  ecatT = _sc_gather_t(cats, tab_t)
  outT = _tc_dense_t(
      ecatT, nums.T, quals.T, W1, b1.reshape(-1, 1), W2, b2.reshape(-1, 1),
      Wq, bq.reshape(-1, 1), gamma.reshape(-1, 1), beta.reshape(-1, 1))
  return outT.T
